# Initial kernel scaffold; baseline (speedup 1.0000x reference)
import jax
import jax.numpy as jnp
from jax.experimental import pallas as pl


def kernel(patch_feats, token_feats, edge_index, W, att_src, att_dst, gat_bias, fc_w, fc_b):
    raise NotImplementedError("write your pallas kernel here")


# speedup vs baseline: 57.6097x; 57.6097x over previous
import functools

import jax
import jax.numpy as jnp
from jax import lax
from jax.experimental import pallas as pl
from jax.experimental.pallas import tpu as pltpu
from jax.experimental.pallas import tpu_sc as plsc

_N = 10000
_E = 160000
_D = 256
_NPAD = 10240
_NC, _NS, _L = 2, 16, 16
_NW = _NC * _NS
_EPW = 5008
_EPAD = _NW * _EPW
_CHUNKS = _EPW // _L
_BLK = 1024


def _proj_body(x_ref, w_ref, c3_ref, at_ref, mx_ref):
    j = pl.program_id(0)
    wc = jnp.dot(w_ref[...], c3_ref[...],
                 preferred_element_type=jnp.float32,
                 precision=lax.Precision.HIGHEST)
    a_t = lax.dot_general(wc, x_ref[...], (((0,), (1,)), ((), ())),
                          preferred_element_type=jnp.float32,
                          precision=lax.Precision.HIGHEST)
    at_ref[...] = a_t
    bs = jnp.max(a_t[0, :])
    bd = jnp.max(a_t[1, :])

    @pl.when(j == 0)
    def _():
        mx_ref[0, 0] = bs
        mx_ref[0, 1] = bd

    @pl.when(j > 0)
    def _():
        mx_ref[0, 0] = jnp.maximum(mx_ref[0, 0], bs)
        mx_ref[0, 1] = jnp.maximum(mx_ref[0, 1], bd)


_proj = pl.pallas_call(
    _proj_body,
    grid=(_NPAD // _BLK,),
    in_specs=[
        pl.BlockSpec((_BLK, _D), lambda j: (j, 0)),
        pl.BlockSpec((_D, _D), lambda j: (0, 0)),
        pl.BlockSpec((_D, 8), lambda j: (0, 0)),
    ],
    out_specs=[
        pl.BlockSpec((8, _BLK), lambda j: (0, j)),
        pl.BlockSpec(block_shape=(1, 2), index_map=lambda j: (0, 0),
                     memory_space=pltpu.SMEM),
    ],
    out_shape=[
        jax.ShapeDtypeStruct((8, _NPAD), jnp.float32),
        jax.ShapeDtypeStruct((1, 2), jnp.float32),
    ],
)


_mesh = plsc.VectorSubcoreMesh(core_axis_name="c", subcore_axis_name="s",
                               num_cores=_NC, num_subcores=_NS)


@functools.partial(
    pl.kernel,
    out_type=[jax.ShapeDtypeStruct((_NW, _NPAD), jnp.float32),
              jax.ShapeDtypeStruct((_NW, _NPAD), jnp.float32)],
    mesh=_mesh,
    compiler_params=pltpu.CompilerParams(needs_layout_passes=False),
    scratch_types=[
        pltpu.VMEM((_EPW,), jnp.int32),
        pltpu.VMEM((_EPW,), jnp.int32),
        pltpu.VMEM((_NPAD,), jnp.float32),
        pltpu.VMEM((_NPAD,), jnp.float32),
        pltpu.VMEM((_NPAD,), jnp.float32),
        pltpu.VMEM((_NPAD,), jnp.float32),
        pltpu.VMEM((_NPAD,), jnp.float32),
        pltpu.VMEM((_L,), jnp.float32),
    ],
)
def _edge_kernel(src_hbm, dst_hbm, asrc_hbm, adst_hbm, p_hbm, m_hbm,
                 s_out, d_out,
                 src_v, dst_v, asrc_v, adst_v, p_v, s_v, d_v, m_v):
    wid = lax.axis_index("s") * _NC + lax.axis_index("c")
    base = wid * _EPW
    pltpu.sync_copy(src_hbm.at[pl.ds(base, _EPW)], src_v)
    pltpu.sync_copy(dst_hbm.at[pl.ds(base, _EPW)], dst_v)
    pltpu.sync_copy(asrc_hbm, asrc_v)
    pltpu.sync_copy(adst_hbm, adst_v)
    pltpu.sync_copy(p_hbm, p_v)
    pltpu.sync_copy(m_hbm, m_v)

    zero = jnp.zeros((_L,), jnp.float32)

    def _zero_body(i, carry):
        s_v[pl.ds(i * _L, _L)] = zero
        d_v[pl.ds(i * _L, _L)] = zero
        return carry

    lax.fori_loop(0, _NPAD // _L, _zero_body, 0)

    m = m_v[...]

    def _body(i, carry):
        si = src_v[pl.ds(i * _L, _L)]
        di = dst_v[pl.ds(i * _L, _L)]
        a1 = plsc.load_gather(asrc_v, [si])
        a2 = plsc.load_gather(adst_v, [di])
        pv = plsc.load_gather(p_v, [si])
        e = a1 + a2
        e = jnp.where(e >= 0.0, e, 0.2 * e)
        al = jnp.exp(e - m)
        plsc.addupdate_scatter(d_v, [di], al)
        plsc.addupdate_scatter(s_v, [di], al * pv)
        return carry

    lax.fori_loop(0, _CHUNKS, _body, 0)

    pltpu.sync_copy(s_v, s_out.at[wid])
    pltpu.sync_copy(d_v, d_out.at[wid])


def _final_body(s_ref, d_ref, c_ref, o_ref):
    s = jnp.sum(s_ref[...], axis=0)
    d = jnp.sum(d_ref[...], axis=0)
    o_ref[...] = (s / (d + 1e-16) + c_ref[0, 0])[None, :]


_final = pl.pallas_call(
    _final_body,
    in_specs=[
        pl.BlockSpec((_NW, _NPAD), lambda: (0, 0)),
        pl.BlockSpec((_NW, _NPAD), lambda: (0, 0)),
        pl.BlockSpec(block_shape=(1, 1), index_map=lambda: (0, 0),
                     memory_space=pltpu.SMEM),
    ],
    out_specs=pl.BlockSpec((1, _NPAD), lambda: (0, 0)),
    out_shape=jax.ShapeDtypeStruct((1, _NPAD), jnp.float32),
)


def kernel(patch_feats, token_feats, edge_index, W, att_src, att_dst,
           gat_bias, fc_w, fc_b):
    x = jnp.concatenate([patch_feats, token_feats], axis=0)
    x_pad = jnp.pad(x, ((0, _NPAD - _N), (0, 0)))
    c3 = jnp.zeros((_D, 8), jnp.float32)
    c3 = c3.at[:, 0].set(att_src[0, 0, :])
    c3 = c3.at[:, 1].set(att_dst[0, 0, :])
    c3 = c3.at[:, 2].set(fc_w[:, 0])

    a_t, mx = _proj(x_pad, W, c3)
    big_m = mx[0, 0] + mx[0, 1]
    big_m = jnp.where(big_m >= 0.0, big_m, 0.2 * big_m)
    m_arr = jnp.full((_L,), big_m, jnp.float32)

    pad_idx = jnp.full((_EPAD - _E,), _N, jnp.int32)
    src_p = jnp.concatenate([edge_index[0], pad_idx])
    dst_p = jnp.concatenate([edge_index[1], pad_idx])

    s_part, d_part = _edge_kernel(src_p, dst_p, a_t[0], a_t[1], a_t[2], m_arr)

    cst = (gat_bias @ fc_w + fc_b).reshape(1, 1)
    out = _final(s_part, d_part, cst)
    return out[0, :_N].reshape(_N, 1)
